# baseline (speedup 1.0000x reference)
import functools
import math

import jax
import jax.numpy as jnp
from jax import lax
from jax.experimental import pallas as pl
from jax.experimental.pallas import tpu as pltpu
from jax.experimental.pallas import tpu_sc as plsc

D_MODEL = 64
SCALE = math.sqrt(D_MODEL)
LANES = 16


@functools.partial(jax.jit, static_argnames=("b", "d", "chunk"))
def _embed_sc(x_flat, table, *, b, d, chunk):
    info = plsc.get_sparse_core_info()
    nc, ns = info.num_cores, info.num_subcores
    nw = nc * ns
    b_per_w = b // nw
    n_chunks = b_per_w // chunk

    mesh = plsc.VectorSubcoreMesh(core_axis_name="c", subcore_axis_name="s")

    @functools.partial(
        pl.kernel,
        mesh=mesh,
        out_type=jax.ShapeDtypeStruct((b, d), jnp.float32),
        scratch_types=[
            pltpu.VMEM((chunk,), jnp.int32),
            pltpu.VMEM((chunk, d), jnp.float32),
            pltpu.SemaphoreType.DMA,
        ],
        compiler_params=pltpu.CompilerParams(use_tc_tiling_on_sc=False),
    )
    def k(x_hbm, table_hbm, out_hbm, idx_v, rows_v, sem):
        wid = lax.axis_index("s") * nc + lax.axis_index("c")
        wbase = wid * b_per_w

        def chunk_body(c, carry):
            base = wbase + c * chunk
            pltpu.sync_copy(x_hbm.at[pl.ds(base, chunk)], idx_v)
            pltpu.async_copy(table_hbm.at[idx_v], rows_v, sem).wait()

            def scale_row(r, carry2):
                for j in range(d // LANES):
                    sl = pl.ds(j * LANES, LANES)
                    rows_v[r, sl] = rows_v[r, sl] * SCALE
                return carry2

            lax.fori_loop(0, chunk, scale_row, 0, unroll=4)
            pltpu.sync_copy(rows_v, out_hbm.at[pl.ds(base, chunk)])
            return carry

        lax.fori_loop(0, n_chunks, chunk_body, 0)

    return k(x_flat, table)


def kernel(x, table):
    s, t = x.shape
    b = s * t
    d = table.shape[1]
    out = _embed_sc(x.reshape(b), table, b=b, d=d, chunk=1024)
    return out.reshape(s, t, d)
